# baseline (speedup 1.0000x reference)
import functools

import jax
import jax.numpy as jnp
from jax import lax
from jax.experimental import pallas as pl
from jax.experimental.pallas import tpu as pltpu
from jax.experimental.pallas import tpu_sc as plsc

DIM = 128
NC, NS = 2, 16
NW = NC * NS
B = 4096 * 50
BPW = B // NW
CH = 128
NCHUNK = BPW // CH
GPM = 2
MCH = GPM * CH
NMAC = BPW // MCH
NBUF = 3

_mesh = plsc.VectorSubcoreMesh(core_axis_name="c", subcore_axis_name="s")


@functools.partial(
    pl.kernel,
    mesh=_mesh,
    out_type=jax.ShapeDtypeStruct((B, DIM), jnp.float32),
    scratch_types=[
        pltpu.VMEM((NCHUNK, CH), jnp.int32),
        pltpu.VMEM((NBUF, MCH, DIM), jnp.float32),
    ]
    + [pltpu.SemaphoreType.DMA] * (2 * NBUF),
)
def _emb_gather(idx_hbm, table_hbm, out_hbm, idx_v, rows_v, *sems):
    gs, os_ = sems[:NBUF], sems[NBUF:]
    wid = lax.axis_index("s") * NC + lax.axis_index("c")
    obase = wid * BPW
    pltpu.sync_copy(idx_hbm.at[wid], idx_v)

    def gstart(j, b):
        for u in range(GPM):
            pltpu.async_copy(
                table_hbm.at[idx_v.at[GPM * j + u]],
                rows_v.at[b, pl.ds(u * CH, CH)],
                gs[b],
            )

    def gwait(j, b):
        for u in range(GPM):
            pltpu.make_async_copy(
                table_hbm.at[idx_v.at[GPM * j + u]],
                rows_v.at[b, pl.ds(u * CH, CH)],
                gs[b],
            ).wait()

    def oslice(j):
        return out_hbm.at[pl.ds(pl.multiple_of(obase + j * MCH, MCH), MCH)]

    def ostart(j, b):
        pltpu.async_copy(rows_v.at[b], oslice(j), os_[b])

    def owait(j, b):
        pltpu.make_async_copy(rows_v.at[b], oslice(j), os_[b]).wait()

    for j in range(NBUF):
        gstart(j, j)
    gwait(0, 0)
    ostart(0, 0)

    def grp(g, carry):
        for u in range(NBUF):
            j = 1 + g * NBUF + u
            b = (1 + u) % NBUF
            bp = (0 + u) % NBUF
            owait(j - 1, bp)

            @pl.when(j + NBUF - 1 < NMAC)
            def _():
                gstart(j + NBUF - 1, bp)

            gwait(j, b)
            ostart(j, b)
        return carry

    lax.fori_loop(0, (NMAC - 1) // NBUF, grp, 0)

    owait(NMAC - 1, (NMAC - 1) % NBUF)


def kernel(inputs, weight):
    n, s = inputs.shape
    idx = inputs.T.reshape(NW, NCHUNK, CH).astype(jnp.int32)
    out = _emb_gather(idx, weight)
    return out.reshape(s, n, DIM).transpose(1, 0, 2)


# speedup vs baseline: 1.0036x; 1.0036x over previous
import functools

import jax
import jax.numpy as jnp
from jax import lax
from jax.experimental import pallas as pl
from jax.experimental.pallas import tpu as pltpu
from jax.experimental.pallas import tpu_sc as plsc

DIM = 128
NC, NS = 2, 16
NW = NC * NS
B = 4096 * 50
BPW = B // NW
CH = 128
NCHUNK = BPW // CH
NBUF = 6

_mesh = plsc.VectorSubcoreMesh(core_axis_name="c", subcore_axis_name="s")


@functools.partial(
    pl.kernel,
    mesh=_mesh,
    out_type=jax.ShapeDtypeStruct((B, DIM), jnp.float32),
    scratch_types=[
        pltpu.VMEM((NCHUNK, CH), jnp.int32),
        pltpu.VMEM((NBUF, CH, DIM), jnp.float32),
    ]
    + [pltpu.SemaphoreType.DMA] * (2 * NBUF),
)
def _emb_gather(idx_hbm, table_hbm, out_hbm, idx_v, rows_v, *sems):
    gs, os_ = sems[:NBUF], sems[NBUF:]
    wid = lax.axis_index("s") * NC + lax.axis_index("c")
    obase = wid * BPW
    pltpu.sync_copy(idx_hbm.at[wid], idx_v)

    def gstart(j, b):
        pltpu.async_copy(table_hbm.at[idx_v.at[j]], rows_v.at[b], gs[b])

    def gwait(j, b):
        pltpu.make_async_copy(
            table_hbm.at[idx_v.at[j]], rows_v.at[b], gs[b]
        ).wait()

    def oslice(j):
        return out_hbm.at[pl.ds(pl.multiple_of(obase + j * CH, CH), CH)]

    def ostart(j, b):
        pltpu.async_copy(rows_v.at[b], oslice(j), os_[b])

    def owait(j, b):
        pltpu.make_async_copy(rows_v.at[b], oslice(j), os_[b]).wait()

    for j in range(NBUF):
        gstart(j, j)
    gwait(0, 0)
    ostart(0, 0)

    def grp(g, carry):
        for u in range(NBUF):
            j = 1 + g * NBUF + u
            b = (1 + u) % NBUF
            bp = (0 + u) % NBUF
            owait(j - 1, bp)

            @pl.when(j + NBUF - 1 < NCHUNK)
            def _():
                gstart(j + NBUF - 1, bp)

            gwait(j, b)
            ostart(j, b)
        return carry

    lax.fori_loop(0, (NCHUNK - 2) // NBUF, grp, 0)

    owait(NCHUNK - 2, (NCHUNK - 2) % NBUF)
    gwait(NCHUNK - 1, (NCHUNK - 1) % NBUF)
    ostart(NCHUNK - 1, (NCHUNK - 1) % NBUF)
    owait(NCHUNK - 1, (NCHUNK - 1) % NBUF)


def kernel(inputs, weight):
    n, s = inputs.shape
    idx = inputs.T.reshape(NW, NCHUNK, CH).astype(jnp.int32)
    out = _emb_gather(idx, weight)
    return out.reshape(s, n, DIM).transpose(1, 0, 2)
